# Initial kernel scaffold; baseline (speedup 1.0000x reference)
import jax
import jax.numpy as jnp
from jax.experimental import pallas as pl


def kernel(boxes, scores, gt_bboxes):
    raise NotImplementedError("write your pallas kernel here")


# speedup vs baseline: 91.4234x; 91.4234x over previous
import functools

import jax
import jax.numpy as jnp
from jax import lax
from jax.experimental import pallas as pl
from jax.experimental.pallas import tpu as pltpu

N = 20000
K = 2000
NUM_GT = 64
NP = 20480
KP = 2048
BLK = 128
NBLK = KP // BLK
NMS_THR = 0.7
MATCH_IOU = 0.5


def _match_body(gt_ref, x1_ref, y1_ref, x2_ref, y2_ref, iou_ref, idx_ref):
    x1 = x1_ref[...]
    y1 = y1_ref[...]
    x2 = x2_ref[...]
    y2 = y2_ref[...]
    area_a = (x2 - x1) * (y2 - y1)

    def body(g, carry):
        best, bidx = carry
        gx1 = gt_ref[g, 0]
        gy1 = gt_ref[g, 1]
        gx2 = gt_ref[g, 2]
        gy2 = gt_ref[g, 3]
        area_b = (gx2 - gx1) * (gy2 - gy1)
        w = jnp.maximum(jnp.minimum(x2, gx2) - jnp.maximum(x1, gx1), 0.0)
        h = jnp.maximum(jnp.minimum(y2, gy2) - jnp.maximum(y1, gy1), 0.0)
        inter = w * h
        union = jnp.maximum(area_a + area_b - inter, 1e-9)
        iou = inter / union
        pred = iou > best
        best = jnp.where(pred, iou, best)
        bidx = jnp.where(pred, g, bidx)
        return best, bidx

    init = (jnp.full(x1.shape, -1.0, jnp.float32),
            jnp.zeros(x1.shape, jnp.int32))
    best, bidx = lax.fori_loop(0, NUM_GT, body, init)
    iou_ref[...] = best
    idx_ref[...] = bidx


def _run_match(gt, bx1, by1, bx2, by2):
    R = NP // 128
    vspec = pl.BlockSpec((R, 128), lambda: (0, 0))
    return pl.pallas_call(
        _match_body,
        grid=(),
        in_specs=[
            pl.BlockSpec(memory_space=pltpu.SMEM),
            vspec, vspec, vspec, vspec,
        ],
        out_specs=[vspec, vspec],
        out_shape=[
            jax.ShapeDtypeStruct((R, 128), jnp.float32),
            jax.ShapeDtypeStruct((R, 128), jnp.int32),
        ],
    )(gt, bx1, by1, bx2, by2)


def _nms_body(x1_ref, y1_ref, x2_ref, y2_ref, keep_ref):
    b = pl.program_id(0)

    @pl.when(b == 0)
    def _():
        keep_ref[...] = jnp.zeros((1, KP), jnp.float32)

    ri = lax.broadcasted_iota(jnp.int32, (BLK, BLK), 0)
    ci = lax.broadcasted_iota(jnp.int32, (BLK, BLK), 1)
    ident = (ri == ci).astype(jnp.float32)
    tri_lt = (ri < ci).astype(jnp.float32)
    tri_gt = (ri > ci).astype(jnp.float32)

    def tcol(v_row):
        return lax.dot_general(ident, v_row, (((1,), (1,)), ((), ())),
                               preferred_element_type=jnp.float32)

    def trow(v_col):
        return lax.dot_general(v_col, ident, (((0,), (0,)), ((), ())),
                               preferred_element_type=jnp.float32)

    s = pl.ds(b * BLK, BLK)
    rx1 = x1_ref[0:1, s]
    ry1 = y1_ref[0:1, s]
    rx2 = x2_ref[0:1, s]
    ry2 = y2_ref[0:1, s]
    cx1 = tcol(rx1)
    cy1 = tcol(ry1)
    cx2 = tcol(rx2)
    cy2 = tcol(ry2)
    area_blk_c = (cx2 - cx1) * (cy2 - cy1)
    area_blk_r = (rx2 - rx1) * (ry2 - ry1)

    ax1 = x1_ref[...]
    ay1 = y1_ref[...]
    ax2 = x2_ref[...]
    ay2 = y2_ref[...]
    area_all = (ax2 - ax1) * (ay2 - ay1)

    def over(u1, v1, u2, v2, w1, z1, w2, z2, area_u, area_w):
        w = jnp.maximum(jnp.minimum(u2, w2) - jnp.maximum(u1, w1), 0.0)
        h = jnp.maximum(jnp.minimum(v2, z2) - jnp.maximum(v1, z1), 0.0)
        inter = w * h
        union = jnp.maximum(area_u + area_w - inter, 1e-9)
        return inter > NMS_THR * union

    s_all = over(cx1, cy1, cx2, cy2, ax1, ay1, ax2, ay2,
                 area_blk_c, area_all)
    colidx = lax.broadcasted_iota(jnp.int32, (1, KP), 1)
    prev = (colidx < b * BLK) & (keep_ref[...] > 0.5)
    sup = jnp.any(s_all & prev, axis=1, keepdims=True)
    sf_col = jnp.where(sup, 0.0, 1.0)
    sf_row = trow(sf_col)

    s_loc = over(cx1, cy1, cx2, cy2, rx1, ry1, rx2, ry2,
                 area_blk_c, area_blk_r).astype(jnp.float32)
    sa = s_loc * sf_col * tri_lt
    sb = s_loc * sf_row * tri_gt

    def cond(carry):
        t, changed, _, _ = carry
        return changed & (t < 66)

    def body(carry):
        t, _, g_col, _ = carry
        g_row2 = 1.0 - jnp.max(sa * g_col, axis=0, keepdims=True)
        g_col2 = 1.0 - jnp.max(sb * g_row2, axis=1, keepdims=True)
        changed = jnp.any(g_col2 != g_col)
        return t + 1, changed, g_col2, g_row2

    init = (jnp.int32(0), True,
            jnp.ones((BLK, 1), jnp.float32), jnp.ones((1, BLK), jnp.float32))
    _, _, _, g_row = lax.while_loop(cond, body, init)
    keep_ref[0:1, s] = sf_row * g_row


def _run_nms(x1, y1, x2, y2):
    vspec = pl.BlockSpec((1, KP), lambda b: (0, 0))
    return pl.pallas_call(
        _nms_body,
        grid=(NBLK,),
        in_specs=[vspec, vspec, vspec, vspec],
        out_specs=vspec,
        out_shape=jax.ShapeDtypeStruct((1, KP), jnp.float32),
    )(x1, y1, x2, y2)


def kernel(boxes, scores, gt_bboxes):
    top_scores, order = lax.top_k(scores, K)
    cand = jnp.take(boxes, order, axis=0)

    candp = jnp.pad(cand, ((0, KP - K), (0, 0)))
    keep = _run_nms(candp[:, 0].reshape(1, KP), candp[:, 1].reshape(1, KP),
                    candp[:, 2].reshape(1, KP), candp[:, 3].reshape(1, KP))
    keepf = keep.reshape(KP)[:K]
    picked_boxes = cand * keepf[:, None]
    picked_scores = top_scores * keepf

    bp = jnp.pad(boxes, ((0, NP - N), (0, 0)))
    R = NP // 128
    best_p, idx_p = _run_match(
        gt_bboxes,
        bp[:, 0].reshape(R, 128), bp[:, 1].reshape(R, 128),
        bp[:, 2].reshape(R, 128), bp[:, 3].reshape(R, 128))
    best_iou = best_p.reshape(NP)[:N]
    best_gt_index = idx_p.reshape(NP)[:N]
    is_foreground = best_iou > MATCH_IOU

    return picked_boxes, picked_scores, best_iou, best_gt_index, is_foreground


# speedup vs baseline: 110.0243x; 1.2035x over previous
import functools

import jax
import jax.numpy as jnp
from jax import lax
from jax.experimental import pallas as pl
from jax.experimental.pallas import tpu as pltpu
from jax.experimental.pallas import tpu_sc as plsc

_NC, _NS = 2, 16
_NW = _NC * _NS

N = 20000
K = 2000
NUM_GT = 64
NP = 20480
KP = 2048
BLK = 128
NBLK = KP // BLK
NMS_THR = 0.7
MATCH_IOU = 0.5


def _match_body(gt_ref, x1_ref, y1_ref, x2_ref, y2_ref, iou_ref, idx_ref):
    x1 = x1_ref[...]
    y1 = y1_ref[...]
    x2 = x2_ref[...]
    y2 = y2_ref[...]
    area_a = (x2 - x1) * (y2 - y1)

    def body(g, carry):
        best, bidx = carry
        gx1 = gt_ref[g, 0]
        gy1 = gt_ref[g, 1]
        gx2 = gt_ref[g, 2]
        gy2 = gt_ref[g, 3]
        area_b = (gx2 - gx1) * (gy2 - gy1)
        w = jnp.maximum(jnp.minimum(x2, gx2) - jnp.maximum(x1, gx1), 0.0)
        h = jnp.maximum(jnp.minimum(y2, gy2) - jnp.maximum(y1, gy1), 0.0)
        inter = w * h
        union = jnp.maximum(area_a + area_b - inter, 1e-9)
        iou = inter / union
        pred = iou > best
        best = jnp.where(pred, iou, best)
        bidx = jnp.where(pred, g, bidx)
        return best, bidx

    init = (jnp.full(x1.shape, -1.0, jnp.float32),
            jnp.zeros(x1.shape, jnp.int32))
    best, bidx = lax.fori_loop(0, NUM_GT, body, init)
    iou_ref[...] = best
    idx_ref[...] = bidx


def _run_match(gt, bx1, by1, bx2, by2):
    R = NP // 128
    vspec = pl.BlockSpec((R, 128), lambda: (0, 0))
    return pl.pallas_call(
        _match_body,
        grid=(),
        in_specs=[
            pl.BlockSpec(memory_space=pltpu.SMEM),
            vspec, vspec, vspec, vspec,
        ],
        out_specs=[vspec, vspec],
        out_shape=[
            jax.ShapeDtypeStruct((R, 128), jnp.float32),
            jax.ShapeDtypeStruct((R, 128), jnp.int32),
        ],
    )(gt, bx1, by1, bx2, by2)


def _nms_body(x1_ref, y1_ref, x2_ref, y2_ref, keep_ref):
    b = pl.program_id(0)

    @pl.when(b == 0)
    def _():
        keep_ref[...] = jnp.zeros((1, KP), jnp.float32)

    ri = lax.broadcasted_iota(jnp.int32, (BLK, BLK), 0)
    ci = lax.broadcasted_iota(jnp.int32, (BLK, BLK), 1)
    ident = (ri == ci).astype(jnp.float32)
    tri_lt = (ri < ci).astype(jnp.float32)
    tri_gt = (ri > ci).astype(jnp.float32)

    def tcol(v_row):
        return lax.dot_general(ident, v_row, (((1,), (1,)), ((), ())),
                               preferred_element_type=jnp.float32)

    def trow(v_col):
        return lax.dot_general(v_col, ident, (((0,), (0,)), ((), ())),
                               preferred_element_type=jnp.float32)

    s = pl.ds(b * BLK, BLK)
    rx1 = x1_ref[0:1, s]
    ry1 = y1_ref[0:1, s]
    rx2 = x2_ref[0:1, s]
    ry2 = y2_ref[0:1, s]
    cx1 = tcol(rx1)
    cy1 = tcol(ry1)
    cx2 = tcol(rx2)
    cy2 = tcol(ry2)
    area_blk_c = (cx2 - cx1) * (cy2 - cy1)
    area_blk_r = (rx2 - rx1) * (ry2 - ry1)

    ax1 = x1_ref[...]
    ay1 = y1_ref[...]
    ax2 = x2_ref[...]
    ay2 = y2_ref[...]
    area_all = (ax2 - ax1) * (ay2 - ay1)

    def over(u1, v1, u2, v2, w1, z1, w2, z2, area_u, area_w):
        w = jnp.maximum(jnp.minimum(u2, w2) - jnp.maximum(u1, w1), 0.0)
        h = jnp.maximum(jnp.minimum(v2, z2) - jnp.maximum(v1, z1), 0.0)
        inter = w * h
        union = jnp.maximum(area_u + area_w - inter, 1e-9)
        return inter > NMS_THR * union

    s_all = over(cx1, cy1, cx2, cy2, ax1, ay1, ax2, ay2,
                 area_blk_c, area_all)
    colidx = lax.broadcasted_iota(jnp.int32, (1, KP), 1)
    prev = (colidx < b * BLK) & (keep_ref[...] > 0.5)
    sup = jnp.any(s_all & prev, axis=1, keepdims=True)
    sf_col = jnp.where(sup, 0.0, 1.0)
    sf_row = trow(sf_col)

    s_loc = over(cx1, cy1, cx2, cy2, rx1, ry1, rx2, ry2,
                 area_blk_c, area_blk_r).astype(jnp.float32)
    sa = s_loc * sf_col * tri_lt
    sb = s_loc * sf_row * tri_gt

    def cond(carry):
        t, changed, _, _ = carry
        return changed & (t < 66)

    def body(carry):
        t, _, g_col, _ = carry
        g_row2 = 1.0 - jnp.max(sa * g_col, axis=0, keepdims=True)
        g_col2 = 1.0 - jnp.max(sb * g_row2, axis=1, keepdims=True)
        changed = jnp.any(g_col2 != g_col)
        return t + 1, changed, g_col2, g_row2

    init = (jnp.int32(0), True,
            jnp.ones((BLK, 1), jnp.float32), jnp.ones((1, BLK), jnp.float32))
    _, _, _, g_row = lax.while_loop(cond, body, init)
    keep_ref[0:1, s] = sf_row * g_row


def _run_nms(x1, y1, x2, y2):
    vspec = pl.BlockSpec((1, KP), lambda b: (0, 0))
    return pl.pallas_call(
        _nms_body,
        grid=(NBLK,),
        in_specs=[vspec, vspec, vspec, vspec],
        out_specs=vspec,
        out_shape=jax.ShapeDtypeStruct((1, KP), jnp.float32),
    )(x1, y1, x2, y2)


def _sc_gather_boxes(bx1, by1, bx2, by2, order_padded):
    rows = KP // _NW
    mesh = plsc.VectorSubcoreMesh(core_axis_name="c", subcore_axis_name="s",
                                  num_cores=_NC, num_subcores=_NS)
    plane = jax.ShapeDtypeStruct((KP,), jnp.float32)

    @functools.partial(
        pl.kernel,
        out_type=(plane, plane, plane, plane),
        mesh=mesh,
        scratch_types=[
            pltpu.VMEM((rows,), jnp.int32),
            [pltpu.VMEM((rows,), jnp.float32)] * 4,
            pltpu.SemaphoreType.DMA,
        ],
    )
    def k(x1_ref, y1_ref, x2_ref, y2_ref, order_ref,
          o1_ref, o2_ref, o3_ref, o4_ref, idx_v, bufs, sem):
        wid = lax.axis_index("s") * _NC + lax.axis_index("c")
        base = wid * rows
        pltpu.sync_copy(order_ref.at[pl.ds(base, rows)], idx_v)
        srcs = (x1_ref, y1_ref, x2_ref, y2_ref)
        outs = (o1_ref, o2_ref, o3_ref, o4_ref)
        descs = [pltpu.async_copy(src.at[idx_v], buf, sem)
                 for src, buf in zip(srcs, bufs)]
        for d in descs:
            d.wait()
        for buf, out in zip(bufs, outs):
            pltpu.sync_copy(buf, out.at[pl.ds(base, rows)])

    return k(bx1, by1, bx2, by2, order_padded)


def kernel(boxes, scores, gt_bboxes):
    top_scores, order = lax.top_k(scores, K)
    order_p = jnp.pad(order, (0, KP - K))
    gx1, gy1, gx2, gy2 = _sc_gather_boxes(
        boxes[:, 0], boxes[:, 1], boxes[:, 2], boxes[:, 3], order_p)
    keep = _run_nms(gx1.reshape(1, KP), gy1.reshape(1, KP),
                    gx2.reshape(1, KP), gy2.reshape(1, KP))
    keepf = keep.reshape(KP)[:K]
    cand = jnp.stack([gx1[:K], gy1[:K], gx2[:K], gy2[:K]], axis=1)
    picked_boxes = cand * keepf[:, None]
    picked_scores = top_scores * keepf

    bp = jnp.pad(boxes, ((0, NP - N), (0, 0)))
    R = NP // 128
    best_p, idx_p = _run_match(
        gt_bboxes,
        bp[:, 0].reshape(R, 128), bp[:, 1].reshape(R, 128),
        bp[:, 2].reshape(R, 128), bp[:, 3].reshape(R, 128))
    best_iou = best_p.reshape(NP)[:N]
    best_gt_index = idx_p.reshape(NP)[:N]
    is_foreground = best_iou > MATCH_IOU

    return picked_boxes, picked_scores, best_iou, best_gt_index, is_foreground
